# baseline (speedup 1.0000x reference)
import functools

import jax
import jax.numpy as jnp
from jax import lax
from jax.experimental import pallas as pl
from jax.experimental.pallas import tpu as pltpu
from jax.experimental.pallas import tpu_sc as plsc

R = 128
N = 100000
L = 16
NC = 2
NS = 16
NW = NC * NS
ROWS_PER_W = R // NW
VREGS_PER_ROW = N // L
CHUNK_VREGS = 1250
CHUNKS_PER_ROW = VREGS_PER_ROW // CHUNK_VREGS


def _sc_body(pred16, tgt, kv, out, buf0, buf1, tgtv, rowv, kvv, outv,
             sem0, sem1, semr):
    wid = lax.axis_index("s") * NC + lax.axis_index("c")
    lanes = lax.iota(jnp.int32, L)

    pltpu.sync_copy(tgt.at[wid], tgtv)
    pltpu.sync_copy(kv.at[0], kvv)
    tv = tgtv[...]
    kthr = lax.reduce_max(kvv[...], axes=(0,))

    bufs = (buf0, buf1)
    sems = (sem0, sem1)

    def chunk_src(j, c):
        row0 = (wid * ROWS_PER_W + j) * VREGS_PER_ROW + c * CHUNK_VREGS
        return pred16.at[pl.ds(row0, CHUNK_VREGS)]

    steps = [(j, c) for j in range(ROWS_PER_W) for c in range(CHUNKS_PER_ROW)]
    copies = {0: pltpu.async_copy(chunk_src(0, 0), bufs[0], sems[0])}

    correct = jnp.int32(0)
    acc = jnp.zeros((L,), jnp.int32)
    idxv = lanes
    v_splat = jnp.zeros((L,), jnp.float32)
    t_splat = jnp.zeros((L,), jnp.int32)

    for s, (j, c) in enumerate(steps):
        p = s % 2
        if s + 1 < len(steps):
            jn, cn = steps[s + 1]
            copies[s + 1] = pltpu.async_copy(chunk_src(jn, cn),
                                             bufs[1 - p], sems[1 - p])
        if c == 0:
            t_scal = lax.reduce_max(
                jnp.where(lanes == j, tv, jnp.int32(0)), axes=(0,))
            g = (wid * ROWS_PER_W + j) * VREGS_PER_ROW + t_scal // L
            pltpu.async_copy(pred16.at[g], rowv, semr).wait()
            rv = rowv[...]
            v_scal = lax.reduce_sum(
                jnp.where(lanes == t_scal % L, rv, jnp.float32(0.0)),
                axes=(0,))
            v_splat = jnp.broadcast_to(v_scal, (L,))
            t_splat = jnp.broadcast_to(t_scal, (L,))
            acc = jnp.zeros((L,), jnp.int32)
            idxv = lanes

        copies[s].wait()
        buf = bufs[p]
        vs, ts = v_splat, t_splat

        def body(i, carry, buf=buf, vs=vs, ts=ts):
            a, iv = carry
            x = buf[i]
            m = (x > vs) | ((x == vs) & (iv < ts))
            a = a + jnp.where(m, jnp.int32(1), jnp.int32(0))
            return a, iv + jnp.int32(L)

        acc, idxv = lax.fori_loop(0, CHUNK_VREGS, body, (acc, idxv))

        if c == CHUNKS_PER_ROW - 1:
            rank = lax.reduce_sum(acc, axes=(0,))
            correct = correct + jnp.where(rank < kthr, jnp.int32(1),
                                          jnp.int32(0))

    outv[...] = jnp.broadcast_to(correct.astype(jnp.float32), (L,))
    pltpu.sync_copy(outv, out.at[wid])


@jax.jit
def _run(pred16, tgt, kv):
    mesh = plsc.VectorSubcoreMesh(core_axis_name="c", subcore_axis_name="s")
    fn = pl.kernel(
        _sc_body,
        out_type=jax.ShapeDtypeStruct((NW, L), jnp.float32),
        mesh=mesh,
        compiler_params=pltpu.CompilerParams(use_tc_tiling_on_sc=False,
                                             needs_layout_passes=False),
        scratch_types=[
            pltpu.VMEM((CHUNK_VREGS, L), jnp.float32),
            pltpu.VMEM((CHUNK_VREGS, L), jnp.float32),
            pltpu.VMEM((L,), jnp.int32),
            pltpu.VMEM((L,), jnp.float32),
            pltpu.VMEM((L,), jnp.int32),
            pltpu.VMEM((L,), jnp.float32),
            pltpu.SemaphoreType.DMA,
            pltpu.SemaphoreType.DMA,
            pltpu.SemaphoreType.DMA,
        ],
    )
    return fn(pred16, tgt, kv)


def kernel(pred, target, k):
    pred16 = pred.reshape(-1, L)
    tgt = jnp.pad(target.astype(jnp.int32).reshape(NW, ROWS_PER_W),
                  ((0, 0), (0, L - ROWS_PER_W)))
    kthr = jnp.minimum(jnp.asarray(k, jnp.int32), 3)
    kv = jnp.broadcast_to(kthr, (1, L))
    partial = _run(pred16, tgt, kv)
    return jnp.sum(partial[:, 0]) / jnp.float32(target.shape[0])


# speedup vs baseline: 1.0277x; 1.0277x over previous
import functools

import jax
import jax.numpy as jnp
from jax import lax
from jax.experimental import pallas as pl
from jax.experimental.pallas import tpu as pltpu
from jax.experimental.pallas import tpu_sc as plsc

R = 128
N = 100000
L = 16
NC = 2
NS = 16
NW = NC * NS
ROWS_PER_W = R // NW
VREGS_PER_ROW = N // L
CHUNK_VREGS = 1250
CHUNKS_PER_ROW = VREGS_PER_ROW // CHUNK_VREGS


def _sc_body(pred16, tgt, kv, out, buf0, buf1, tgtv, rowv, kvv, outv,
             sem0, sem1, semr):
    wid = lax.axis_index("s") * NC + lax.axis_index("c")
    lanes = lax.iota(jnp.int32, L)

    pltpu.sync_copy(tgt.at[wid], tgtv)
    pltpu.sync_copy(kv.at[0], kvv)
    tv = tgtv[...]
    kthr = lax.reduce_max(kvv[...], axes=(0,))

    bufs = (buf0, buf1)
    sems = (sem0, sem1)

    def chunk_src(j, c):
        row0 = (wid * ROWS_PER_W + j) * VREGS_PER_ROW + c * CHUNK_VREGS
        return pred16.at[pl.ds(row0, CHUNK_VREGS)]

    steps = [(j, c) for j in range(ROWS_PER_W) for c in range(CHUNKS_PER_ROW)]
    copies = {0: pltpu.async_copy(chunk_src(0, 0), bufs[0], sems[0])}

    correct = jnp.int32(0)
    acc = jnp.zeros((L,), jnp.int32)
    idxv = lanes
    v_splat = jnp.zeros((L,), jnp.float32)
    t_splat = jnp.zeros((L,), jnp.int32)

    for s, (j, c) in enumerate(steps):
        p = s % 2
        if s + 1 < len(steps):
            jn, cn = steps[s + 1]
            copies[s + 1] = pltpu.async_copy(chunk_src(jn, cn),
                                             bufs[1 - p], sems[1 - p])
        if c == 0:
            t_scal = lax.reduce_max(
                jnp.where(lanes == j, tv, jnp.int32(0)), axes=(0,))
            g = (wid * ROWS_PER_W + j) * VREGS_PER_ROW + t_scal // L
            pltpu.async_copy(pred16.at[g], rowv, semr).wait()
            rv = rowv[...]
            v_scal = lax.reduce_sum(
                jnp.where(lanes == t_scal % L, rv, jnp.float32(0.0)),
                axes=(0,))
            v_splat = jnp.broadcast_to(v_scal, (L,))
            t_splat = jnp.broadcast_to(t_scal, (L,))
            acc = jnp.zeros((L,), jnp.int32)
            idxv = lanes

        copies[s].wait()
        buf = bufs[p]
        vs, ts = v_splat, t_splat

        @plsc.parallel_loop(0, CHUNK_VREGS, 1, unroll=8, carry=(acc, idxv))
        def _loop(i, carry, buf=buf, vs=vs, ts=ts):
            a, iv = carry
            x = buf[i]
            m = (x > vs) | ((x == vs) & (iv < ts))
            a = a + jnp.where(m, jnp.int32(1), jnp.int32(0))
            return a, iv + jnp.int32(L)

        acc, idxv = _loop

        if c == CHUNKS_PER_ROW - 1:
            rank = lax.reduce_sum(acc, axes=(0,))
            correct = correct + jnp.where(rank < kthr, jnp.int32(1),
                                          jnp.int32(0))

    outv[...] = jnp.broadcast_to(correct.astype(jnp.float32), (L,))
    pltpu.sync_copy(outv, out.at[wid])


@jax.jit
def _run(pred16, tgt, kv):
    mesh = plsc.VectorSubcoreMesh(core_axis_name="c", subcore_axis_name="s")
    fn = pl.kernel(
        _sc_body,
        out_type=jax.ShapeDtypeStruct((NW, L), jnp.float32),
        mesh=mesh,
        compiler_params=pltpu.CompilerParams(use_tc_tiling_on_sc=False,
                                             needs_layout_passes=False),
        scratch_types=[
            pltpu.VMEM((CHUNK_VREGS, L), jnp.float32),
            pltpu.VMEM((CHUNK_VREGS, L), jnp.float32),
            pltpu.VMEM((L,), jnp.int32),
            pltpu.VMEM((L,), jnp.float32),
            pltpu.VMEM((L,), jnp.int32),
            pltpu.VMEM((L,), jnp.float32),
            pltpu.SemaphoreType.DMA,
            pltpu.SemaphoreType.DMA,
            pltpu.SemaphoreType.DMA,
        ],
    )
    return fn(pred16, tgt, kv)


def kernel(pred, target, k):
    pred16 = pred.reshape(-1, L)
    tgt = jnp.pad(target.astype(jnp.int32).reshape(NW, ROWS_PER_W),
                  ((0, 0), (0, L - ROWS_PER_W)))
    kthr = jnp.minimum(jnp.asarray(k, jnp.int32), 3)
    kv = jnp.broadcast_to(kthr, (1, L))
    partial = _run(pred16, tgt, kv)
    return jnp.sum(partial[:, 0]) / jnp.float32(target.shape[0])


# speedup vs baseline: 1.9021x; 1.8509x over previous
import functools

import jax
import jax.numpy as jnp
from jax import lax
from jax.experimental import pallas as pl
from jax.experimental.pallas import tpu as pltpu
from jax.experimental.pallas import tpu_sc as plsc

R = 128
N = 100000
L = 16
NC = 2
NS = 16
NW = NC * NS
RG = 8
NG = R // RG
TILE = 128
FULL_TILES = N // TILE
TAIL = N - FULL_TILES * TILE
H0_TILES = 390
H1_START = 391
CHUNK_TILES = 30
NCHUNKS = 13
CHUNK_COLS = CHUNK_TILES * TILE
CHUNK_VREGS = CHUNK_COLS // L
EXTRA_BASE0 = H0_TILES * TILE
EXTRA_BASE1 = FULL_TILES * TILE
H_COL0 = EXTRA_BASE1 - EXTRA_BASE0


def _stage1(pred, extra, meta, outp, buf0, buf1, gbuf, ebuf, metav, outv,
            sem0, sem1, semg):
    wid = lax.axis_index("s") * NC + lax.axis_index("c")
    g = wid // 2
    h = wid % 2
    lanes = lax.iota(jnp.int32, L)

    pltpu.sync_copy(meta, metav)

    def chunk_src(c):
        col = pl.multiple_of((h * H1_START + c * CHUNK_TILES) * TILE, TILE)
        return pred.at[pl.ds(g * RG, RG), pl.ds(col, CHUNK_COLS)]

    bufs = (buf0, buf1)
    sems = (sem0, sem1)
    copies = {0: pltpu.async_copy(chunk_src(0), bufs[0], sems[0])}

    t_splat, v_splat, acc = [], [], []
    for j in range(RG):
        row = g * RG + j
        tv = metav[0, pl.ds(pl.multiple_of((row // L) * L, L), L)]
        t_scal = lax.reduce_max(
            jnp.where(lanes == row % L, tv, jnp.int32(0)), axes=(0,))
        gcol = pl.multiple_of((t_scal // TILE) * TILE, TILE)
        pltpu.async_copy(
            pred.at[pl.ds(g * RG, RG), pl.ds(gcol, TILE)], gbuf, semg
        ).wait()
        sub = pl.multiple_of(((t_scal % TILE) // L) * L, L)
        rv = gbuf[j, pl.ds(sub, L)]
        v_scal = lax.reduce_sum(
            jnp.where(lanes == t_scal % L, rv, jnp.float32(0.0)), axes=(0,))
        t_splat.append(jnp.broadcast_to(t_scal, (L,)))
        v_splat.append(jnp.broadcast_to(v_scal, (L,)))
        acc.append(jnp.zeros((L,), jnp.int32))

    col_base = h * H_COL0

    def count_block(buf, ncols, cb, accs):
        iv0 = jnp.broadcast_to(cb, (L,)) + lanes

        @plsc.parallel_loop(0, ncols // L, 1, carry=tuple(accs) + (iv0,))
        def _loop(i, carry):
            a = list(carry[:RG])
            iv = carry[RG]
            off = i * L
            for j in range(RG):
                x = buf[j, pl.ds(off, L)]
                m = (x > v_splat[j]) | ((x == v_splat[j]) & (iv < t_splat[j]))
                a[j] = a[j] + jnp.where(m, jnp.int32(1), jnp.int32(0))
            return tuple(a) + (iv + jnp.int32(L),)

        return list(_loop[:RG])

    for c in range(NCHUNKS):
        p = c % 2
        if c + 1 < NCHUNKS:
            copies[c + 1] = pltpu.async_copy(chunk_src(c + 1),
                                             bufs[1 - p], sems[1 - p])
        copies[c].wait()
        acc = count_block(bufs[p], CHUNK_COLS,
                          col_base + c * CHUNK_COLS, acc)

    ecol = pl.multiple_of(h * TILE, TILE)
    pltpu.async_copy(extra.at[pl.ds(g * RG, RG), pl.ds(ecol, TILE)],
                     ebuf, semg).wait()
    acc = count_block(ebuf, TILE, EXTRA_BASE0 + h * H_COL0, acc)

    for j in range(RG):
        rank = lax.reduce_sum(acc[j], axes=(0,))
        outv[j, pl.ds(0, L)] = jnp.broadcast_to(rank, (L,))
    ocol = pl.multiple_of(wid * TILE, TILE)
    pltpu.sync_copy(outv, outp.at[pl.ds(0, RG), pl.ds(ocol, TILE)])


def _stage2(outp, meta, out, pv, metav, outv, semc):
    wid = lax.axis_index("s") * NC + lax.axis_index("c")
    lanes = lax.iota(jnp.int32, L)

    @pl.when(wid == 0)
    def _():
        pltpu.sync_copy(outp, pv)
        pltpu.sync_copy(meta, metav)
        kv = metav[1, pl.ds(0, L)]
        kthr = lax.reduce_max(kv, axes=(0,))
        total = jnp.zeros((L,), jnp.int32)
        for rb in range(R // L):
            rows = jnp.int32(rb * L) + lanes
            jvec = rows % RG
            w0 = (rows // RG) * 2
            r0 = plsc.load_gather(pv, [jvec, w0 * TILE])
            r1 = plsc.load_gather(pv, [jvec, w0 * TILE + TILE])
            rank = r0 + r1
            total = total + jnp.where(rank < jnp.broadcast_to(kthr, (L,)),
                                      jnp.int32(1), jnp.int32(0))
        correct = lax.reduce_sum(total, axes=(0,))
        outv[0, pl.ds(0, L)] = jnp.broadcast_to(
            correct.astype(jnp.float32), (L,))
        for j in range(1, RG):
            outv[j, pl.ds(0, L)] = jnp.zeros((L,), jnp.float32)
        pltpu.sync_copy(outv, out)


@jax.jit
def _run(pred, extra, meta):
    mesh = plsc.VectorSubcoreMesh(core_axis_name="c", subcore_axis_name="s")
    params = pltpu.CompilerParams(needs_layout_passes=False)
    s1 = pl.kernel(
        _stage1,
        out_type=jax.ShapeDtypeStruct((RG, NW * TILE), jnp.int32),
        mesh=mesh,
        compiler_params=params,
        scratch_types=[
            pltpu.VMEM((RG, CHUNK_COLS), jnp.float32),
            pltpu.VMEM((RG, CHUNK_COLS), jnp.float32),
            pltpu.VMEM((RG, TILE), jnp.float32),
            pltpu.VMEM((RG, TILE), jnp.float32),
            pltpu.VMEM((RG, TILE), jnp.int32),
            pltpu.VMEM((RG, TILE), jnp.int32),
            pltpu.SemaphoreType.DMA,
            pltpu.SemaphoreType.DMA,
            pltpu.SemaphoreType.DMA,
        ],
    )
    outp = s1(pred, extra, meta)
    s2 = pl.kernel(
        _stage2,
        out_type=jax.ShapeDtypeStruct((RG, TILE), jnp.float32),
        mesh=mesh,
        compiler_params=params,
        scratch_types=[
            pltpu.VMEM((RG, NW * TILE), jnp.int32),
            pltpu.VMEM((RG, TILE), jnp.int32),
            pltpu.VMEM((RG, TILE), jnp.float32),
            pltpu.SemaphoreType.DMA,
        ],
    )
    return s2(outp, meta)


def kernel(pred, target, k):
    kthr = jnp.minimum(jnp.asarray(k, jnp.int32), 3)
    meta = jnp.zeros((RG, TILE), jnp.int32)
    meta = meta.at[0].set(target.astype(jnp.int32))
    meta = meta.at[1].set(jnp.broadcast_to(kthr, (TILE,)))
    extra = jnp.concatenate(
        [pred[:, EXTRA_BASE0:EXTRA_BASE0 + TILE],
         pred[:, EXTRA_BASE1:],
         jnp.full((R, TILE - TAIL), -jnp.inf, jnp.float32)], axis=1)
    out = _run(pred, extra, meta)
    return out[0, 0] / jnp.float32(target.shape[0])


# speedup vs baseline: 1.9057x; 1.0019x over previous
import functools

import jax
import jax.numpy as jnp
from jax import lax
from jax.experimental import pallas as pl
from jax.experimental.pallas import tpu as pltpu
from jax.experimental.pallas import tpu_sc as plsc

R = 128
N = 100000
L = 16
NC = 2
NS = 16
NW = NC * NS
RG = 8
NG = R // RG
TILE = 128
FULL_TILES = N // TILE
TAIL = N - FULL_TILES * TILE
H0_TILES = 390
H1_START = 391
CHUNK_TILES = 30
NCHUNKS = 13
CHUNK_COLS = CHUNK_TILES * TILE
CHUNK_VREGS = CHUNK_COLS // L
EXTRA_BASE0 = H0_TILES * TILE
EXTRA_BASE1 = FULL_TILES * TILE
H_COL0 = EXTRA_BASE1 - EXTRA_BASE0


def _stage1(pred, extra, meta, outp, buf0, buf1, gbuf, ebuf, metav, outv,
            sem0, sem1, semg):
    wid = lax.axis_index("s") * NC + lax.axis_index("c")
    g = wid // 2
    h = wid % 2
    lanes = lax.iota(jnp.int32, L)

    pltpu.sync_copy(meta, metav)

    def chunk_src(c):
        col = pl.multiple_of((h * H1_START + c * CHUNK_TILES) * TILE, TILE)
        return pred.at[pl.ds(g * RG, RG), pl.ds(col, CHUNK_COLS)]

    bufs = (buf0, buf1)
    sems = (sem0, sem1)
    copies = {0: pltpu.async_copy(chunk_src(0), bufs[0], sems[0])}

    t_splat, v_splat, acc = [], [], []
    for j in range(RG):
        row = g * RG + j
        tv = metav[0, pl.ds(pl.multiple_of((row // L) * L, L), L)]
        t_scal = lax.reduce_max(
            jnp.where(lanes == row % L, tv, jnp.int32(0)), axes=(0,))
        gcol = pl.multiple_of((t_scal // TILE) * TILE, TILE)
        pltpu.async_copy(
            pred.at[pl.ds(g * RG, RG), pl.ds(gcol, TILE)], gbuf, semg
        ).wait()
        sub = pl.multiple_of(((t_scal % TILE) // L) * L, L)
        rv = gbuf[j, pl.ds(sub, L)]
        v_scal = lax.reduce_sum(
            jnp.where(lanes == t_scal % L, rv, jnp.float32(0.0)), axes=(0,))
        t_splat.append(jnp.broadcast_to(t_scal, (L,)))
        v_splat.append(jnp.broadcast_to(v_scal, (L,)))
        acc.append(jnp.zeros((L,), jnp.int32))

    col_base = h * H_COL0

    def count_block(buf, ncols, cb, accs):
        iv0 = jnp.broadcast_to(cb, (L,)) + lanes

        @plsc.parallel_loop(0, ncols // L, 1, carry=tuple(accs) + (iv0,))
        def _loop(i, carry):
            a = list(carry[:RG])
            iv = carry[RG]
            off = i * L
            for j in range(RG):
                x = buf[j, pl.ds(off, L)]
                m = (x > v_splat[j]) | ((x == v_splat[j]) & (iv < t_splat[j]))
                a[j] = a[j] + jnp.where(m, jnp.int32(1), jnp.int32(0))
            return tuple(a) + (iv + jnp.int32(L),)

        return list(_loop[:RG])

    for c in range(NCHUNKS):
        p = c % 2
        if c + 1 < NCHUNKS:
            copies[c + 1] = pltpu.async_copy(chunk_src(c + 1),
                                             bufs[1 - p], sems[1 - p])
        copies[c].wait()
        acc = count_block(bufs[p], CHUNK_COLS,
                          col_base + c * CHUNK_COLS, acc)

    ecol = pl.multiple_of(h * TILE, TILE)
    pltpu.async_copy(extra.at[pl.ds(g * RG, RG), pl.ds(ecol, TILE)],
                     ebuf, semg).wait()
    acc = count_block(ebuf, TILE, EXTRA_BASE0 + h * H_COL0, acc)

    for j in range(RG):
        rank = lax.reduce_sum(acc[j], axes=(0,))
        outv[j, pl.ds(0, L)] = jnp.broadcast_to(rank, (L,))
    ocol = pl.multiple_of(wid * TILE, TILE)
    pltpu.sync_copy(outv, outp.at[pl.ds(0, RG), pl.ds(ocol, TILE)])


def _stage2(outp, meta, out, pv, metav, outv, semc):
    wid = lax.axis_index("s") * NC + lax.axis_index("c")
    lanes = lax.iota(jnp.int32, L)

    @pl.when(wid == 0)
    def _():
        pltpu.sync_copy(outp, pv)
        pltpu.sync_copy(meta, metav)
        kv = metav[1, pl.ds(0, L)]
        kthr = lax.reduce_max(kv, axes=(0,))
        total = jnp.zeros((L,), jnp.int32)
        for rb in range(R // L):
            rows = jnp.int32(rb * L) + lanes
            jvec = rows % RG
            w0 = (rows // RG) * 2
            r0 = plsc.load_gather(pv, [jvec, w0 * TILE])
            r1 = plsc.load_gather(pv, [jvec, w0 * TILE + TILE])
            rank = r0 + r1
            total = total + jnp.where(rank < jnp.broadcast_to(kthr, (L,)),
                                      jnp.int32(1), jnp.int32(0))
        correct = lax.reduce_sum(total, axes=(0,))
        outv[0, pl.ds(0, L)] = jnp.broadcast_to(
            correct.astype(jnp.float32), (L,))
        for j in range(1, RG):
            outv[j, pl.ds(0, L)] = jnp.zeros((L,), jnp.float32)
        pltpu.sync_copy(outv, out)


@jax.jit
def _run(pred, extra, meta):
    mesh = plsc.VectorSubcoreMesh(core_axis_name="c", subcore_axis_name="s")
    params = pltpu.CompilerParams(needs_layout_passes=False,
                                  use_tc_tiling_on_sc=True)
    s1 = pl.kernel(
        _stage1,
        out_type=jax.ShapeDtypeStruct((RG, NW * TILE), jnp.int32),
        mesh=mesh,
        compiler_params=params,
        scratch_types=[
            pltpu.VMEM((RG, CHUNK_COLS), jnp.float32),
            pltpu.VMEM((RG, CHUNK_COLS), jnp.float32),
            pltpu.VMEM((RG, TILE), jnp.float32),
            pltpu.VMEM((RG, TILE), jnp.float32),
            pltpu.VMEM((RG, TILE), jnp.int32),
            pltpu.VMEM((RG, TILE), jnp.int32),
            pltpu.SemaphoreType.DMA,
            pltpu.SemaphoreType.DMA,
            pltpu.SemaphoreType.DMA,
        ],
    )
    outp = s1(pred, extra, meta)
    s2 = pl.kernel(
        _stage2,
        out_type=jax.ShapeDtypeStruct((RG, TILE), jnp.float32),
        mesh=mesh,
        compiler_params=params,
        scratch_types=[
            pltpu.VMEM((RG, NW * TILE), jnp.int32),
            pltpu.VMEM((RG, TILE), jnp.int32),
            pltpu.VMEM((RG, TILE), jnp.float32),
            pltpu.SemaphoreType.DMA,
        ],
    )
    return s2(outp, meta)


def kernel(pred, target, k):
    kthr = jnp.minimum(jnp.asarray(k, jnp.int32), 3)
    meta = jnp.zeros((RG, TILE), jnp.int32)
    meta = meta.at[0].set(target.astype(jnp.int32))
    meta = meta.at[1].set(jnp.broadcast_to(kthr, (TILE,)))
    extra = jnp.concatenate(
        [pred[:, EXTRA_BASE0:EXTRA_BASE0 + TILE],
         pred[:, EXTRA_BASE1:],
         jnp.full((R, TILE - TAIL), -jnp.inf, jnp.float32)], axis=1)
    out = _run(pred, extra, meta)
    return out[0, 0] / jnp.float32(target.shape[0])


# speedup vs baseline: 3.3761x; 1.7716x over previous
import jax
import jax.numpy as jnp
from jax import lax
from jax.experimental import pallas as pl
from jax.experimental.pallas import tpu as pltpu
from jax.experimental.pallas import tpu_sc as plsc

R = 128
N = 100000
L = 16
NC = 2
NS = 16
NW = NC * NS
NB = R // L
SUB = 8
TILES = N // SUB
W_TILES = TILES // NW
REM = TILES - W_TILES * NW
CHUNK_TILES = 30
NCHUNKS = W_TILES // CHUNK_TILES
CHUNK_ROWS = CHUNK_TILES * SUB
NEG_TINY_BITS = -2147483647


def _stage1(pt, meta, tgt, outp, buf0, buf1, gath, metav, tgtv, ebuf, outv,
            sem0, sem1, semg):
    wid = lax.axis_index("s") * NC + lax.axis_index("c")
    lanes = lax.iota(jnp.int32, L) + wid * jnp.int32(0)

    pltpu.sync_copy(meta, metav)
    pltpu.sync_copy(tgt, tgtv)
    pltpu.async_copy(pt.at[tgtv], gath, semg).wait()

    t_vec, thr_lo, thr_hi, acc = [], [], [], []
    for m in range(NB):
        v = jnp.broadcast_to(jnp.float32(0.0), (L,)) + lanes.astype(
            jnp.float32) * jnp.float32(0.0)
        for l in range(L):
            row = gath[m * L + l, pl.ds(m * L, L)]
            v = jnp.where(lanes == jnp.int32(l), row, v)
        b = lax.bitcast_convert_type(v, jnp.int32)
        blo = jnp.where(v > jnp.float32(0.0), b - 1, b + 1)
        blo = jnp.where(v == jnp.float32(0.0), jnp.int32(NEG_TINY_BITS), blo)
        thr_lo.append(lax.bitcast_convert_type(blo, jnp.float32))
        thr_hi.append(v)
        t_vec.append(metav[0, pl.ds(m * L, L)])
        acc.append(lanes * jnp.int32(0))

    def chunk_src(c):
        return pt.at[pl.ds((wid * W_TILES + c * CHUNK_TILES) * SUB,
                           CHUNK_ROWS), pl.ds(0, R)]

    bufs = (buf0, buf1)
    sems = (sem0, sem1)
    copies = {0: pltpu.async_copy(chunk_src(0), bufs[0], sems[0])}

    def count_block(buf, nrows, col0, accs, lo, hi):
        cs0 = jnp.broadcast_to(col0, (L,))

        @plsc.parallel_loop(0, nrows, 1, carry=tuple(accs) + (cs0,))
        def _loop(s, carry):
            a = list(carry[:NB])
            cs = carry[NB]
            for m in range(NB):
                x = buf[s, pl.ds(m * L, L)]
                thr = jnp.where(cs < t_vec[m], lo[m], hi[m])
                a[m] = a[m] + jnp.where(x > thr, jnp.int32(1), jnp.int32(0))
            return tuple(a) + (cs + jnp.int32(1),)

        return list(_loop[:NB])

    for c in range(NCHUNKS):
        p = c % 2
        if c + 1 < NCHUNKS:
            copies[c + 1] = pltpu.async_copy(chunk_src(c + 1),
                                             bufs[1 - p], sems[1 - p])
        copies[c].wait()
        acc = count_block(bufs[p], CHUNK_ROWS,
                          (wid * W_TILES + c * CHUNK_TILES) * SUB, acc,
                          thr_lo, thr_hi)

    en = jnp.broadcast_to(wid < REM, (L,))
    row0 = (NW * W_TILES + jnp.minimum(wid, REM - 1)) * SUB
    pltpu.async_copy(pt.at[pl.ds(row0, SUB), pl.ds(0, R)],
                     ebuf, semg).wait()
    inf = jnp.float32(float("inf"))
    lo_e = [jnp.where(en, thr_lo[m], inf) for m in range(NB)]
    hi_e = [jnp.where(en, thr_hi[m], inf) for m in range(NB)]
    acc = count_block(ebuf, SUB, row0, acc, lo_e, hi_e)

    for m in range(NB):
        outv[0, pl.ds(m * L, L)] = acc[m]
    pltpu.sync_copy(outv, outp.at[pl.ds(0, SUB),
                                  pl.ds(pl.multiple_of(wid * R, R), R)])


def _stage2(outp, meta, out, pv, metav, outv, semc):
    wid = lax.axis_index("s") * NC + lax.axis_index("c")
    lanes = lax.iota(jnp.int32, L) + wid * jnp.int32(0)

    @pl.when(wid == 0)
    def _():
        pltpu.sync_copy(outp, pv)
        pltpu.sync_copy(meta, metav)
        kthr = jnp.broadcast_to(
            lax.reduce_max(metav[1, pl.ds(0, L)], axes=(0,)), (L,))
        correct = jnp.zeros((L,), jnp.int32)
        for m in range(NB):
            rank = pv[0, pl.ds(m * L, L)]
            for w in range(1, NW):
                rank = rank + pv[0, pl.ds(w * R + m * L, L)]
            correct = correct + jnp.where(rank < kthr, jnp.int32(1),
                                          jnp.int32(0))
        total = lax.reduce_sum(correct, axes=(0,))
        outv[0, pl.ds(0, L)] = jnp.broadcast_to(
            total.astype(jnp.float32), (L,))
        for j in range(1, SUB):
            outv[j, pl.ds(0, L)] = jnp.zeros((L,), jnp.float32)
        pltpu.sync_copy(outv, out)


@jax.jit
def _run(pt, meta, tgt):
    mesh = plsc.VectorSubcoreMesh(core_axis_name="c", subcore_axis_name="s")
    params = pltpu.CompilerParams(needs_layout_passes=False,
                                  use_tc_tiling_on_sc=True)
    s1 = pl.kernel(
        _stage1,
        out_type=jax.ShapeDtypeStruct((SUB, NW * R), jnp.int32),
        mesh=mesh,
        compiler_params=params,
        scratch_types=[
            pltpu.VMEM((CHUNK_ROWS, R), jnp.float32),
            pltpu.VMEM((CHUNK_ROWS, R), jnp.float32),
            pltpu.VMEM((R, R), jnp.float32),
            pltpu.VMEM((SUB, R), jnp.int32),
            pltpu.VMEM((R,), jnp.int32),
            pltpu.VMEM((SUB, R), jnp.float32),
            pltpu.VMEM((SUB, R), jnp.int32),
            pltpu.SemaphoreType.DMA,
            pltpu.SemaphoreType.DMA,
            pltpu.SemaphoreType.DMA,
        ],
    )
    outp = s1(pt, meta, tgt)
    s2 = pl.kernel(
        _stage2,
        out_type=jax.ShapeDtypeStruct((SUB, R), jnp.float32),
        mesh=mesh,
        compiler_params=params,
        scratch_types=[
            pltpu.VMEM((SUB, NW * R), jnp.int32),
            pltpu.VMEM((SUB, R), jnp.int32),
            pltpu.VMEM((SUB, R), jnp.float32),
            pltpu.SemaphoreType.DMA,
        ],
    )
    return s2(outp, meta)


def kernel(pred, target, k):
    tgt = target.astype(jnp.int32)
    kthr = jnp.minimum(jnp.asarray(k, jnp.int32), 3)
    meta = jnp.zeros((SUB, R), jnp.int32)
    meta = meta.at[0].set(tgt)
    meta = meta.at[1].set(jnp.broadcast_to(kthr, (R,)))
    out = _run(pred.T, meta, tgt)
    return out[0, 0] / jnp.float32(target.shape[0])
